# Initial kernel scaffold; baseline (speedup 1.0000x reference)
import jax
import jax.numpy as jnp
from jax.experimental import pallas as pl


def kernel(logits, output_tokens, presence_penalties, frequency_penalties, temperatures, top_ps, top_ks):
    raise NotImplementedError("write your pallas kernel here")


# speedup vs baseline: 6.8054x; 6.8054x over previous
import functools

import jax
import jax.numpy as jnp
from jax import lax
from jax.experimental import pallas as pl
from jax.experimental.pallas import tpu as pltpu
from jax.experimental.pallas import tpu_sc as plsc

BN = 128
V = 100000
VPAD = 100096
HIST = 200
HPAD = 208
NEG = -1e30


SC_CHUNK = 25024
SC_NCHUNK = VPAD // SC_CHUNK


def _sc_body(logits_hbm, toks_hbm, freq_hbm, pres_hbm, out_hbm,
             buf, tokvm, fvm, pvm):
    nc = 2
    ns = 16
    wid = lax.axis_index("s") * nc + lax.axis_index("c")
    rows_per = BN // (nc * ns)

    pltpu.sync_copy(freq_hbm, fvm)
    pltpu.sync_copy(pres_hbm, pvm)

    iota16 = lax.iota(jnp.int32, 16)

    for r in range(rows_per):
        row = wid * rows_per + r
        row_vec = jnp.full((16,), row, dtype=jnp.int32)
        fneg = -plsc.load_gather(fvm, [row_vec])
        pneg = -plsc.load_gather(pvm, [row_vec])
        pltpu.sync_copy(toks_hbm.at[row], tokvm.at[pl.ds(0, HIST)])

        for c in range(SC_NCHUNK):
            base = c * SC_CHUNK
            real = min(SC_CHUNK, V - base)
            pltpu.sync_copy(logits_hbm.at[row, pl.ds(base, real)],
                            buf.at[pl.ds(0, real)])
            if real < SC_CHUNK:
                padv = jnp.full((16,), NEG, dtype=jnp.float32)
                for i in range((SC_CHUNK - real) // 16):
                    buf[pl.ds(real + 16 * i, 16)] = padv

            idxs, msks = [], []
            for v in range(HPAD // 16):
                tv = tokvm[pl.ds(v * 16, 16)]
                li = tv - base
                valid = iota16 < (HIST - v * 16)
                m = (li >= 0) & (li < real) & valid
                idxs.append(jnp.where(m, li, 0))
                msks.append(m)
            vals = [plsc.load_gather(buf, [idxs[v]], mask=msks[v])
                    for v in range(HPAD // 16)]
            for v in range(HPAD // 16):
                plsc.store_scatter(buf, [idxs[v]], vals[v] + pneg,
                                   mask=msks[v])
            for v in range(HPAD // 16):
                plsc.addupdate_scatter(buf, [idxs[v]], fneg, mask=msks[v])

            pltpu.sync_copy(buf.at[pl.ds(0, SC_CHUNK)],
                            out_hbm.at[row, pl.ds(base, SC_CHUNK)])


def _sc_penalize(logits, toks, freq, pres):
    mesh = plsc.VectorSubcoreMesh(core_axis_name="c", subcore_axis_name="s")
    fn = pl.kernel(
        _sc_body,
        out_type=jax.ShapeDtypeStruct((BN, VPAD), jnp.float32),
        mesh=mesh,
        scratch_types=[
            pltpu.VMEM((SC_CHUNK,), jnp.float32),
            pltpu.VMEM((HPAD,), jnp.int32),
            pltpu.VMEM((BN,), jnp.float32),
            pltpu.VMEM((BN,), jnp.float32),
        ],
        compiler_params=pltpu.CompilerParams(use_tc_tiling_on_sc=False,
                                             needs_layout_passes=False),
    )
    return fn(logits, toks, freq, pres)


R = 8
TCH = 4352
NTCH = VPAD // TCH
N_FULL_OUT = V // TCH
TAIL = V - N_FULL_OUT * TCH


def _tc_body(x_ref, t_ref, p_ref, k_ref, probs_ref, nt_ref, lp_ref, e_ref):
    inv_t = 1.0 / jnp.maximum(t_ref[:, 0:1], 0.05)
    kf = jnp.maximum(k_ref[:, 0:1], 1).astype(jnp.float32)
    pc = jnp.clip(p_ref[:, 0:1], 0.05, 1.0)

    def pha(i, carry):
        m, bi = carry
        st = pl.multiple_of(i * TCH, 128)
        s = x_ref[:, pl.ds(st, TCH)] * inv_t
        cm = jnp.max(s, axis=1, keepdims=True)
        ii = lax.broadcasted_iota(jnp.int32, (R, TCH), 1) + st
        ci = jnp.min(jnp.where(s == cm, ii, jnp.int32(2**30)),
                     axis=1, keepdims=True)
        upd = cm > m
        return jnp.where(upd, cm, m), jnp.where(upd, ci, bi)

    m, bi = lax.fori_loop(0, NTCH, pha, (
        jnp.full((R, 1), NEG, jnp.float32), jnp.zeros((R, 1), jnp.int32)))

    def phb(i, z):
        st = pl.multiple_of(i * TCH, 128)
        s = x_ref[:, pl.ds(st, TCH)] * inv_t
        ec = jnp.exp(s - m)
        e_ref[:, pl.ds(st, TCH)] = ec
        return z + jnp.sum(ec, axis=1, keepdims=True)

    z = lax.fori_loop(0, NTCH, phb, jnp.zeros((R, 1), jnp.float32))
    pz = pc * z

    def phc(_, lohi):
        lo, hi = lohi
        mid = (lo + hi) >> 1
        t = lax.bitcast_convert_type(mid, jnp.float32)

        def inner(i, cm_):
            cnt, mass = cm_
            st = pl.multiple_of(i * TCH, 128)
            ec = e_ref[:, pl.ds(st, TCH)]
            gt = ec > t
            cnt = cnt + jnp.sum(gt.astype(jnp.float32), axis=1, keepdims=True)
            mass = mass + jnp.sum(jnp.where(gt, ec, 0.0), axis=1,
                                  keepdims=True)
            return cnt, mass

        cnt, mass = lax.fori_loop(0, NTCH, inner, (
            jnp.zeros((R, 1), jnp.float32), jnp.zeros((R, 1), jnp.float32)))
        pred = (cnt >= kf) | (mass > pz)
        return jnp.where(pred, mid + 1, lo), jnp.where(pred, hi, mid)

    one_f = jnp.full((R, 1), 0x3F800000, jnp.int32)
    lo, hi = lax.fori_loop(0, 31, phc, (jnp.zeros((R, 1), jnp.int32), one_f))
    tau = lax.bitcast_convert_type(hi, jnp.float32)

    def phd(i, s_):
        st = pl.multiple_of(i * TCH, 128)
        ec = e_ref[:, pl.ds(st, TCH)]
        return s_ + jnp.sum(jnp.where(ec >= tau, ec, 0.0), axis=1,
                            keepdims=True)

    skept = lax.fori_loop(0, NTCH, phd, jnp.zeros((R, 1), jnp.float32))
    inv_s = 1.0 / skept

    def phe(i, _):
        st = pl.multiple_of(i * TCH, 128)
        ec = e_ref[:, pl.ds(st, TCH)]
        probs_ref[:, pl.ds(st, TCH)] = jnp.where(ec >= tau, ec * inv_s, 0.0)
        return 0

    lax.fori_loop(0, N_FULL_OUT, phe, 0)
    ec = e_ref[:, pl.ds(N_FULL_OUT * TCH, TAIL)]
    probs_ref[:, pl.ds(N_FULL_OUT * TCH, TAIL)] = jnp.where(
        ec >= tau, ec * inv_s, 0.0)

    nt_ref[...] = jnp.broadcast_to(bi, (R, 128))
    lp_ref[...] = jnp.broadcast_to(-jnp.log(z), (R, 128))


def _tc_sample(xpen, temps, tops, topk):
    grid = (BN // R,)
    out = pl.pallas_call(
        _tc_body,
        grid=grid,
        in_specs=[
            pl.BlockSpec((R, VPAD), lambda i: (i, 0)),
            pl.BlockSpec((R, 128), lambda i: (i, 0)),
            pl.BlockSpec((R, 128), lambda i: (i, 0)),
            pl.BlockSpec((R, 128), lambda i: (i, 0)),
        ],
        out_specs=[
            pl.BlockSpec((R, V), lambda i: (i, 0)),
            pl.BlockSpec((R, 128), lambda i: (i, 0)),
            pl.BlockSpec((R, 128), lambda i: (i, 0)),
        ],
        out_shape=[
            jax.ShapeDtypeStruct((BN, V), jnp.float32),
            jax.ShapeDtypeStruct((BN, 128), jnp.int32),
            jax.ShapeDtypeStruct((BN, 128), jnp.float32),
        ],
        scratch_shapes=[pltpu.VMEM((R, VPAD), jnp.float32)],
        compiler_params=pltpu.CompilerParams(
            dimension_semantics=("parallel",),
            vmem_limit_bytes=100 * 1024 * 1024,
        ),
    )(xpen, temps, tops, topk)
    return out


def kernel(logits, output_tokens, presence_penalties, frequency_penalties,
           temperatures, top_ps, top_ks):
    xpen = _sc_penalize(logits, output_tokens, frequency_penalties,
                        presence_penalties)
    t2 = jnp.broadcast_to(temperatures[:, None], (BN, 128))
    p2 = jnp.broadcast_to(top_ps[:, None], (BN, 128))
    k2 = jnp.broadcast_to(top_ks[:, None], (BN, 128))
    probs, nt, lp = _tc_sample(xpen, t2, p2, k2)
    return nt[:, 0], probs, lp[:, 0]
